# baseline (speedup 1.0000x reference)
import functools
import math

import jax
import jax.numpy as jnp
from jax import lax
from jax.experimental import pallas as pl
from jax.experimental.pallas import tpu as pltpu
from jax.experimental.pallas import tpu_sc as plsc

D = 128
M = 16
C = 256
ROPE_BASE = 10000.0


def _rot_table_body(emb_ref, out_ref):
    m = pl.program_id(0).astype(jnp.float32)
    e = emb_ref[...]
    col = lax.broadcasted_iota(jnp.int32, (1, D), 1)
    u = col % 32
    ncol = (col - u) + (u % 2) * 16 + u // 2
    two_k = (ncol - (ncol % 2)).astype(jnp.float32)
    freq = jnp.exp(two_k * (-math.log(ROPE_BASE) / D))
    ang = m * freq
    cosr = jnp.cos(ang)
    sinr = jnp.sin(ang) * jnp.where((ncol % 2) == 1, 1.0, -1.0)
    rows = lax.broadcasted_iota(jnp.int32, (D, D), 0)
    p1 = (rows == ncol).astype(jnp.float32)
    p2 = (rows == (ncol ^ 1)).astype(jnp.float32)
    ep = jnp.dot(e, p1, preferred_element_type=jnp.float32)
    es = jnp.dot(e, p2, preferred_element_type=jnp.float32)
    out_ref[...] = (ep * cosr + es * sinr).astype(jnp.bfloat16)


def _rot_table(char_emb):
    return pl.pallas_call(
        _rot_table_body,
        grid=(M,),
        in_specs=[pl.BlockSpec((C, D), lambda m: (0, 0))],
        out_specs=pl.BlockSpec((C, D), lambda m: (m, 0)),
        out_shape=jax.ShapeDtypeStruct((M * C, D), jnp.bfloat16),
    )(char_emb)


_SC_PARAMS = dict(
    compiler_params=pltpu.CompilerParams(use_tc_tiling_on_sc=False,
                                         needs_layout_passes=False,
                                         disable_bounds_checks=True),
)


def _sc_info():
    info = plsc.get_sparse_core_info()
    return plsc.VectorSubcoreMesh(core_axis_name="c", subcore_axis_name="s"), info


def _sc_charsum(ids, char_table, rot):
    b, s = ids.shape
    n = b * s
    mesh, info = _sc_info()
    nw = info.num_cores * info.num_subcores
    per_w = n // nw
    wpr = s // per_w
    T = 16
    nchunk = per_w // T
    NS = T * M // 128

    @functools.partial(
        pl.kernel,
        out_type=jax.ShapeDtypeStruct((b, s, D), jnp.float32),
        mesh=mesh,
        scratch_types=[
            pltpu.VMEM((per_w,), jnp.int32),
            [pltpu.VMEM((T * M,), jnp.int32)] * 2,
            [pltpu.VMEM((T * M,), jnp.int32)] * 2,
            [pltpu.VMEM((T * M,), jnp.int32)] * 2,
            [pltpu.VMEM((T, D), jnp.float32)] * 2,
            [pltpu.VMEM((T * M, D), jnp.bfloat16)] * 2,
            [pltpu.SemaphoreType.DMA] * 2,
            [pltpu.SemaphoreType.DMA] * 2,
        ],
        **_SC_PARAMS,
    )
    def k(ids_hbm, chart_hbm, rot_hbm, out_hbm,
          ids_v, cidx, chars, rc, acc, rb, sem_c, sem_r):
        wid = lax.axis_index("s") * info.num_cores + lax.axis_index("c")
        row = wid // wpr
        s0 = (wid % wpr) * per_w
        pltpu.sync_copy(ids_hbm.at[row, pl.ds(s0, per_w)], ids_v)
        lane = lax.iota(jnp.int32, 16)
        offs = lane * C

        def fire_char(c, p):
            v = ids_v[pl.ds(c * T, 16)] * M
            for t in range(16):
                cidx[p][pl.ds(t * M, M)] = v[t] + lane
            for i in range(NS):
                sl = pl.ds(i * 128, 128)
                pltpu.async_copy(chart_hbm.at[cidx[p].at[sl]], chars[p].at[sl],
                                 sem_c[p])

        def fire_rot(c, p):
            for i in range(NS):
                sl = pl.ds(i * 128, 128)
                pltpu.make_async_copy(chart_hbm.at[cidx[p].at[sl]],
                                      chars[p].at[sl], sem_c[p]).wait()
            for v in range(T * M // 16):
                sl = pl.ds(v * 16, 16)
                rc[p][sl] = chars[p][sl] + offs
            for i in range(NS):
                sl = pl.ds(i * 128, 128)
                pltpu.async_copy(rot_hbm.at[rc[p].at[sl]], rb[p].at[sl],
                                 sem_r[p])

        def drain_accum(c, p):
            for i in range(NS):
                sl = pl.ds(i * 128, 128)
                pltpu.make_async_copy(rot_hbm.at[rc[p].at[sl]], rb[p].at[sl],
                                      sem_r[p]).wait()

            def tok_body(j, carry2):
                rbase = j * M
                for g in range(D // 32):
                    raw = rb[p][rbase, pl.ds(g * 32, 32)]
                    a, bb = plsc.unpack(raw,
                                        format=plsc.PackFormat.INTERLEAVED)
                    for m in range(1, M):
                        raw = rb[p][rbase + m, pl.ds(g * 32, 32)]
                        x, y = plsc.unpack(raw,
                                           format=plsc.PackFormat.INTERLEAVED)
                        a = a + x
                        bb = bb + y
                    acc[p][j, pl.ds(g * 32, 16)] = a
                    acc[p][j, pl.ds(g * 32 + 16, 16)] = bb
                return carry2

            lax.fori_loop(0, T, tok_body, 0)
            pltpu.sync_copy(acc[p], out_hbm.at[row, pl.ds(s0 + c * T, T), :])

        fire_char(0, 0)
        fire_char(1, 1)
        fire_rot(0, 0)

        def pair_body(c2, carry):
            c = c2 * 2
            for p in (0, 1):
                cc = c + p
                q = 1 - p
                fire_rot(cc + 1, q)
                fire_char(cc + 2, p)
                drain_accum(cc, p)
            return carry

        lax.fori_loop(0, (nchunk - 2) // 2, pair_body, 0)

        fire_rot(nchunk - 1, 1)
        drain_accum(nchunk - 2, 0)
        drain_accum(nchunk - 1, 1)

    return k(ids, char_table, rot)


def _sc_tokadd(ids, tok_emb, partial):
    b, s = ids.shape
    n = b * s
    mesh, info = _sc_info()
    nw = info.num_cores * info.num_subcores
    per_w = n // nw
    wpr = s // per_w
    T2 = 128
    nchunk = per_w // T2

    @functools.partial(
        pl.kernel,
        out_type=jax.ShapeDtypeStruct((b, s, D), jnp.float32),
        mesh=mesh,
        scratch_types=[
            pltpu.VMEM((per_w,), jnp.int32),
            [pltpu.VMEM((T2, D), jnp.float32)] * 2,
            pltpu.VMEM((T2, D), jnp.float32),
            [pltpu.SemaphoreType.DMA] * 2,
        ],
        **_SC_PARAMS,
    )
    def k(ids_hbm, tok_hbm, part_hbm, out_hbm, ids_v, tk, pr, sem_t):
        wid = lax.axis_index("s") * info.num_cores + lax.axis_index("c")
        row = wid // wpr
        s0 = (wid % wpr) * per_w
        pltpu.sync_copy(ids_hbm.at[row, pl.ds(s0, per_w)], ids_v)
        pltpu.async_copy(tok_hbm.at[ids_v.at[pl.ds(0, T2)]], tk[0], sem_t[0])
        for c in range(nchunk):
            p = c % 2
            if c + 1 < nchunk:
                pltpu.async_copy(tok_hbm.at[ids_v.at[pl.ds((c + 1) * T2, T2)]],
                                 tk[1 - p], sem_t[1 - p])
            pltpu.sync_copy(part_hbm.at[row, pl.ds(s0 + c * T2, T2), :], pr)
            pltpu.make_async_copy(tok_hbm.at[ids_v.at[pl.ds(c * T2, T2)]],
                                  tk[p], sem_t[p]).wait()

            def add_body(j, carry):
                for kk in range(D // 16):
                    sl = pl.ds(kk * 16, 16)
                    pr[j, sl] = pr[j, sl] + tk[p][j, sl]
                return carry

            lax.fori_loop(0, T2, add_body, 0)
            pltpu.sync_copy(pr, out_hbm.at[row, pl.ds(s0 + c * T2, T2), :])

    return k(ids, tok_emb, partial)


def kernel(input, char_table, char_emb, tok_emb):
    rot = _rot_table(char_emb)
    partial = _sc_charsum(input, char_table.reshape(-1), rot)
    return _sc_tokadd(input, tok_emb, partial)


# speedup vs baseline: 1.0085x; 1.0085x over previous
import functools
import math

import jax
import jax.numpy as jnp
from jax import lax
from jax.experimental import pallas as pl
from jax.experimental.pallas import tpu as pltpu
from jax.experimental.pallas import tpu_sc as plsc

D = 128
M = 16
C = 256
ROPE_BASE = 10000.0


def _rot_table_body(emb_ref, out_ref):
    m = pl.program_id(0).astype(jnp.float32)
    e = emb_ref[...]
    col = lax.broadcasted_iota(jnp.int32, (1, D), 1)
    u = col % 32
    ncol = (col - u) + (u % 2) * 16 + u // 2
    two_k = (ncol - (ncol % 2)).astype(jnp.float32)
    freq = jnp.exp(two_k * (-math.log(ROPE_BASE) / D))
    ang = m * freq
    cosr = jnp.cos(ang)
    sinr = jnp.sin(ang) * jnp.where((ncol % 2) == 1, 1.0, -1.0)
    rows = lax.broadcasted_iota(jnp.int32, (D, D), 0)
    p1 = (rows == ncol).astype(jnp.float32)
    p2 = (rows == (ncol ^ 1)).astype(jnp.float32)
    ep = jnp.dot(e, p1, preferred_element_type=jnp.float32)
    es = jnp.dot(e, p2, preferred_element_type=jnp.float32)
    out_ref[...] = (ep * cosr + es * sinr).astype(jnp.bfloat16)


def _rot_table(char_emb):
    return pl.pallas_call(
        _rot_table_body,
        grid=(M,),
        in_specs=[pl.BlockSpec((C, D), lambda m: (0, 0))],
        out_specs=pl.BlockSpec((C, D), lambda m: (m, 0)),
        out_shape=jax.ShapeDtypeStruct((M * C, D), jnp.bfloat16),
    )(char_emb)


_SC_PARAMS = dict(
    compiler_params=pltpu.CompilerParams(use_tc_tiling_on_sc=False,
                                         needs_layout_passes=False,
                                         disable_bounds_checks=True),
)


def _sc_info():
    info = plsc.get_sparse_core_info()
    return plsc.VectorSubcoreMesh(core_axis_name="c", subcore_axis_name="s"), info


def _sc_charsum(ids, char_table, rot):
    b, s = ids.shape
    n = b * s
    mesh, info = _sc_info()
    nw = info.num_cores * info.num_subcores
    per_w = n // nw
    wpr = s // per_w
    T = 16
    nchunk = per_w // T
    NS = T * M // 128

    @functools.partial(
        pl.kernel,
        out_type=jax.ShapeDtypeStruct((b, s, D), jnp.float32),
        mesh=mesh,
        scratch_types=[
            pltpu.VMEM((per_w,), jnp.int32),
            [pltpu.VMEM((T * M,), jnp.int32)] * 2,
            [pltpu.VMEM((T * M,), jnp.int32)] * 2,
            [pltpu.VMEM((T * M,), jnp.int32)] * 2,
            [pltpu.VMEM((T, D), jnp.float32)] * 2,
            [pltpu.VMEM((T * M, D), jnp.bfloat16)] * 2,
            [pltpu.SemaphoreType.DMA] * 2,
            [pltpu.SemaphoreType.DMA] * 2,
        ],
        **_SC_PARAMS,
    )
    def k(ids_hbm, chart_hbm, rot_hbm, out_hbm,
          ids_v, cidx, chars, rc, acc, rb, sem_c, sem_r):
        wid = lax.axis_index("s") * info.num_cores + lax.axis_index("c")
        row = wid // wpr
        s0 = (wid % wpr) * per_w
        pltpu.sync_copy(ids_hbm.at[row, pl.ds(s0, per_w)], ids_v)
        lane = lax.iota(jnp.int32, 16)
        offs = lane * C

        def fire_char(c, p):
            v = ids_v[pl.ds(c * T, 16)] * M
            for t in range(16):
                cidx[p][pl.ds(t * M, M)] = v[t] + lane
            for i in range(NS):
                sl = pl.ds(i * 128, 128)
                pltpu.async_copy(chart_hbm.at[cidx[p].at[sl]], chars[p].at[sl],
                                 sem_c[p])

        def fire_rot(c, p):
            for i in range(NS):
                sl = pl.ds(i * 128, 128)
                pltpu.make_async_copy(chart_hbm.at[cidx[p].at[sl]],
                                      chars[p].at[sl], sem_c[p]).wait()
            for v in range(T * M // 16):
                sl = pl.ds(v * 16, 16)
                rc[p][sl] = chars[p][sl] + offs
            for i in range(NS):
                sl = pl.ds(i * 128, 128)
                pltpu.async_copy(rot_hbm.at[rc[p].at[sl]], rb[p].at[sl],
                                 sem_r[p])

        def drain_accum(c, p):
            for i in range(NS):
                sl = pl.ds(i * 128, 128)
                pltpu.make_async_copy(rot_hbm.at[rc[p].at[sl]], rb[p].at[sl],
                                      sem_r[p]).wait()

            def tok_body(j, carry2):
                rbase = j * M
                for g in range(D // 32):
                    raw = rb[p][rbase, pl.ds(g * 32, 32)]
                    a, bb = plsc.unpack(raw,
                                        format=plsc.PackFormat.INTERLEAVED)
                    for m in range(1, M):
                        raw = rb[p][rbase + m, pl.ds(g * 32, 32)]
                        x, y = plsc.unpack(raw,
                                           format=plsc.PackFormat.INTERLEAVED)
                        a = a + x
                        bb = bb + y
                    acc[p][j, pl.ds(g * 32, 16)] = a
                    acc[p][j, pl.ds(g * 32 + 16, 16)] = bb
                return carry2

            lax.fori_loop(0, T, tok_body, 0)
            pltpu.sync_copy(acc[p], out_hbm.at[row, pl.ds(s0 + c * T, T), :])

        fire_char(0, 0)
        fire_char(1, 1)
        fire_rot(0, 0)

        def pair_body(c2, carry):
            c = c2 * 2
            for p in (0, 1):
                cc = c + p
                q = 1 - p
                fire_rot(cc + 1, q)
                fire_char(cc + 2, p)
                drain_accum(cc, p)
            return carry

        lax.fori_loop(0, (nchunk - 2) // 2, pair_body, 0)

        fire_rot(nchunk - 1, 1)
        drain_accum(nchunk - 2, 0)
        drain_accum(nchunk - 1, 1)

    return k(ids, char_table, rot)


def _sc_tokadd(ids, tok_emb, partial):
    b, s = ids.shape
    n = b * s
    mesh, info = _sc_info()
    nw = info.num_cores * info.num_subcores
    per_w = n // nw
    wpr = s // per_w
    T2 = 128
    nchunk = per_w // T2

    @functools.partial(
        pl.kernel,
        out_type=jax.ShapeDtypeStruct((b, s, D), jnp.float32),
        mesh=mesh,
        scratch_types=[
            pltpu.VMEM((per_w,), jnp.int32),
            [pltpu.VMEM((T2, D), jnp.float32)] * 2,
            pltpu.VMEM((T2, D), jnp.float32),
            [pltpu.SemaphoreType.DMA] * 2,
        ],
        compiler_params=pltpu.CompilerParams(needs_layout_passes=False,
                                             disable_bounds_checks=True),
    )
    def k(ids_hbm, tok_hbm, part_hbm, out_hbm, ids_v, tk, pr, sem_t):
        wid = lax.axis_index("s") * info.num_cores + lax.axis_index("c")
        row = wid // wpr
        s0 = (wid % wpr) * per_w
        pltpu.sync_copy(ids_hbm.at[row, pl.ds(s0, per_w)], ids_v)
        pltpu.async_copy(tok_hbm.at[ids_v.at[pl.ds(0, T2)]], tk[0], sem_t[0])
        for c in range(nchunk):
            p = c % 2
            if c + 1 < nchunk:
                pltpu.async_copy(tok_hbm.at[ids_v.at[pl.ds((c + 1) * T2, T2)]],
                                 tk[1 - p], sem_t[1 - p])
            pltpu.sync_copy(part_hbm.at[row, pl.ds(s0 + c * T2, T2), :], pr)
            pltpu.make_async_copy(tok_hbm.at[ids_v.at[pl.ds(c * T2, T2)]],
                                  tk[p], sem_t[p]).wait()

            def add_body(j, carry):
                for kk in range(D // 16):
                    sl = pl.ds(kk * 16, 16)
                    pr[j, sl] = pr[j, sl] + tk[p][j, sl]
                return carry

            lax.fori_loop(0, T2, add_body, 0)
            pltpu.sync_copy(pr, out_hbm.at[row, pl.ds(s0 + c * T2, T2), :])

    return k(ids, tok_emb, partial)


def kernel(input, char_table, char_emb, tok_emb):
    rot = _rot_table(char_emb)
    partial = _sc_charsum(input, char_table.reshape(-1), rot)
    return _sc_tokadd(input, tok_emb, partial)


# speedup vs baseline: 1.2078x; 1.1976x over previous
import functools
import math

import jax
import jax.numpy as jnp
from jax import lax
from jax.experimental import pallas as pl
from jax.experimental.pallas import tpu as pltpu
from jax.experimental.pallas import tpu_sc as plsc

D = 128
M = 16
C = 256
ROPE_BASE = 10000.0


def _rot_table_body(emb_ref, out_ref):
    m = pl.program_id(0).astype(jnp.float32)
    e = emb_ref[...]
    col = lax.broadcasted_iota(jnp.int32, (1, D), 1)
    u = col % 32
    ncol = (col - u) + (u % 2) * 16 + u // 2
    two_k = (ncol - (ncol % 2)).astype(jnp.float32)
    freq = jnp.exp(two_k * (-math.log(ROPE_BASE) / D))
    ang = m * freq
    cosr = jnp.cos(ang)
    sinr = jnp.sin(ang) * jnp.where((ncol % 2) == 1, 1.0, -1.0)
    rows = lax.broadcasted_iota(jnp.int32, (D, D), 0)
    p1 = (rows == ncol).astype(jnp.float32)
    p2 = (rows == (ncol ^ 1)).astype(jnp.float32)
    ep = jnp.dot(e, p1, preferred_element_type=jnp.float32)
    es = jnp.dot(e, p2, preferred_element_type=jnp.float32)
    out_ref[...] = (ep * cosr + es * sinr).astype(jnp.bfloat16)


def _rot_table(char_emb):
    return pl.pallas_call(
        _rot_table_body,
        grid=(M,),
        in_specs=[pl.BlockSpec((C, D), lambda m: (0, 0))],
        out_specs=pl.BlockSpec((C, D), lambda m: (m, 0)),
        out_shape=jax.ShapeDtypeStruct((M * C, D), jnp.bfloat16),
    )(char_emb)


_SC_PARAMS = dict(
    compiler_params=pltpu.CompilerParams(use_tc_tiling_on_sc=False,
                                         needs_layout_passes=False,
                                         disable_bounds_checks=True),
)


def _sc_info():
    info = plsc.get_sparse_core_info()
    return plsc.VectorSubcoreMesh(core_axis_name="c", subcore_axis_name="s"), info


def _sc_charsum(ids, char_table, rot):
    b, s = ids.shape
    n = b * s
    mesh, info = _sc_info()
    nw = info.num_cores * info.num_subcores
    per_w = n // nw
    wpr = s // per_w
    T = 16
    nchunk = per_w // T
    NS = T * M // 128
    V = char_table.shape[0] // M

    @functools.partial(
        pl.kernel,
        out_type=jax.ShapeDtypeStruct((b, s, D), jnp.float32),
        mesh=mesh,
        scratch_types=[
            pltpu.VMEM((per_w,), jnp.int32),
            [pltpu.VMEM((T * M,), jnp.int32)] * 2,
            [pltpu.VMEM((T * M,), jnp.int32)] * 2,
            [pltpu.VMEM((T * M,), jnp.int32)] * 2,
            [pltpu.VMEM((T, D), jnp.float32)] * 2,
            [pltpu.VMEM((T * M, D), jnp.bfloat16)] * 2,
            [pltpu.SemaphoreType.DMA] * 2,
            [pltpu.SemaphoreType.DMA] * 2,
        ],
        **_SC_PARAMS,
    )
    def k(ids_hbm, chart_hbm, rot_hbm, out_hbm,
          ids_v, cidx, chars, rc, acc, rb, sem_c, sem_r):
        wid = lax.axis_index("s") * info.num_cores + lax.axis_index("c")
        row = wid // wpr
        s0 = (wid % wpr) * per_w
        pltpu.sync_copy(ids_hbm.at[row, pl.ds(s0, per_w)], ids_v)

        def fire_char(c, p):
            v = ids_v[pl.ds(c * T, T)]
            for m in range(M):
                cidx[p][pl.ds(m * T, T)] = v + m * V
            for i in range(NS):
                sl = pl.ds(i * 128, 128)
                pltpu.async_copy(chart_hbm.at[cidx[p].at[sl]], chars[p].at[sl],
                                 sem_c[p])

        def fire_rot(c, p):
            for i in range(NS):
                sl = pl.ds(i * 128, 128)
                pltpu.make_async_copy(chart_hbm.at[cidx[p].at[sl]],
                                      chars[p].at[sl], sem_c[p]).wait()
            for m in range(M):
                sl = pl.ds(m * T, T)
                rc[p][sl] = chars[p][sl] + m * C
            for i in range(NS):
                sl = pl.ds(i * 128, 128)
                pltpu.async_copy(rot_hbm.at[rc[p].at[sl]], rb[p].at[sl],
                                 sem_r[p])

        def drain_accum(c, p):
            for i in range(NS):
                sl = pl.ds(i * 128, 128)
                pltpu.make_async_copy(rot_hbm.at[rc[p].at[sl]], rb[p].at[sl],
                                      sem_r[p]).wait()

            def tok_body(j, carry2):
                for g in range(D // 32):
                    raw = rb[p][j, pl.ds(g * 32, 32)]
                    a, bb = plsc.unpack(raw,
                                        format=plsc.PackFormat.INTERLEAVED)
                    for m in range(1, M):
                        raw = rb[p][m * T + j, pl.ds(g * 32, 32)]
                        x, y = plsc.unpack(raw,
                                           format=plsc.PackFormat.INTERLEAVED)
                        a = a + x
                        bb = bb + y
                    acc[p][j, pl.ds(g * 32, 16)] = a
                    acc[p][j, pl.ds(g * 32 + 16, 16)] = bb
                return carry2

            lax.fori_loop(0, T, tok_body, 0)
            pltpu.sync_copy(acc[p], out_hbm.at[row, pl.ds(s0 + c * T, T), :])

        fire_char(0, 0)
        fire_char(1, 1)
        fire_rot(0, 0)

        def pair_body(c2, carry):
            c = c2 * 2
            for p in (0, 1):
                cc = c + p
                q = 1 - p
                fire_rot(cc + 1, q)
                fire_char(cc + 2, p)
                drain_accum(cc, p)
            return carry

        lax.fori_loop(0, (nchunk - 2) // 2, pair_body, 0)

        fire_rot(nchunk - 1, 1)
        drain_accum(nchunk - 2, 0)
        drain_accum(nchunk - 1, 1)

    return k(ids, char_table, rot)


def _sc_tokadd(ids, tok_emb, partial):
    b, s = ids.shape
    n = b * s
    mesh, info = _sc_info()
    nw = info.num_cores * info.num_subcores
    per_w = n // nw
    wpr = s // per_w
    T2 = 128
    nchunk = per_w // T2

    @functools.partial(
        pl.kernel,
        out_type=jax.ShapeDtypeStruct((b, s, D), jnp.float32),
        mesh=mesh,
        scratch_types=[
            pltpu.VMEM((per_w,), jnp.int32),
            [pltpu.VMEM((T2, D), jnp.float32)] * 2,
            pltpu.VMEM((T2, D), jnp.float32),
            [pltpu.SemaphoreType.DMA] * 2,
        ],
        compiler_params=pltpu.CompilerParams(needs_layout_passes=False,
                                             disable_bounds_checks=True),
    )
    def k(ids_hbm, tok_hbm, part_hbm, out_hbm, ids_v, tk, pr, sem_t):
        wid = lax.axis_index("s") * info.num_cores + lax.axis_index("c")
        row = wid // wpr
        s0 = (wid % wpr) * per_w
        pltpu.sync_copy(ids_hbm.at[row, pl.ds(s0, per_w)], ids_v)
        pltpu.async_copy(tok_hbm.at[ids_v.at[pl.ds(0, T2)]], tk[0], sem_t[0])
        for c in range(nchunk):
            p = c % 2
            if c + 1 < nchunk:
                pltpu.async_copy(tok_hbm.at[ids_v.at[pl.ds((c + 1) * T2, T2)]],
                                 tk[1 - p], sem_t[1 - p])
            pltpu.sync_copy(part_hbm.at[row, pl.ds(s0 + c * T2, T2), :], pr)
            pltpu.make_async_copy(tok_hbm.at[ids_v.at[pl.ds(c * T2, T2)]],
                                  tk[p], sem_t[p]).wait()

            def add_body(j, carry):
                for kk in range(D // 16):
                    sl = pl.ds(kk * 16, 16)
                    pr[j, sl] = pr[j, sl] + tk[p][j, sl]
                return carry

            lax.fori_loop(0, T2, add_body, 0)
            pltpu.sync_copy(pr, out_hbm.at[row, pl.ds(s0 + c * T2, T2), :])

    return k(ids, tok_emb, partial)


def kernel(input, char_table, char_emb, tok_emb):
    rot = _rot_table(char_emb)
    partial = _sc_charsum(input, char_table.T.reshape(-1), rot)
    return _sc_tokadd(input, tok_emb, partial)


# speedup vs baseline: 1.5143x; 1.2538x over previous
import functools
import math

import jax
import jax.numpy as jnp
from jax import lax
from jax.experimental import pallas as pl
from jax.experimental.pallas import tpu as pltpu
from jax.experimental.pallas import tpu_sc as plsc

D = 128
M = 16
C = 256
ROPE_BASE = 10000.0


def _rot_table_body(emb_ref, out_ref):
    e = emb_ref[...]
    col = lax.broadcasted_iota(jnp.int32, (1, D), 1)
    u = col % 32
    ncol = (col - u) + (u % 2) * 16 + u // 2
    two_k = (ncol - (ncol % 2)).astype(jnp.float32)
    freq = jnp.exp(two_k * (-math.log(ROPE_BASE) / D))
    sign = jnp.where((ncol % 2) == 1, 1.0, -1.0)
    rows = lax.broadcasted_iota(jnp.int32, (D, D), 0)
    p1 = (rows == ncol).astype(jnp.float32)
    p2 = (rows == (ncol ^ 1)).astype(jnp.float32)
    ep = jnp.dot(e, p1, preferred_element_type=jnp.float32)
    es = jnp.dot(e, p2, preferred_element_type=jnp.float32)
    for m in range(M):
        ang = m * freq
        cosr = jnp.cos(ang)
        sinr = jnp.sin(ang) * sign
        out_ref[pl.ds(m * C, C), :] = (ep * cosr + es * sinr).astype(
            jnp.bfloat16)


def _rot_table(char_emb):
    return pl.pallas_call(
        _rot_table_body,
        out_shape=jax.ShapeDtypeStruct((M * C, D), jnp.bfloat16),
    )(char_emb)


_SC_PARAMS = dict(
    compiler_params=pltpu.CompilerParams(use_tc_tiling_on_sc=False,
                                         needs_layout_passes=False,
                                         disable_bounds_checks=True),
)


def _sc_info():
    info = plsc.get_sparse_core_info()
    return plsc.VectorSubcoreMesh(core_axis_name="c", subcore_axis_name="s"), info


def _sc_charsum(ids, char_table, rot, tok_emb):
    b, s = ids.shape
    n = b * s
    mesh, info = _sc_info()
    nw = info.num_cores * info.num_subcores
    per_w = n // nw
    wpr = s // per_w
    T = 16
    nchunk = per_w // T
    NS = T * M // 128
    V = char_table.shape[0] // M

    @functools.partial(
        pl.kernel,
        out_type=jax.ShapeDtypeStruct((b, s, D), jnp.float32),
        mesh=mesh,
        scratch_types=[
            pltpu.VMEM((per_w,), jnp.int32),
            [pltpu.VMEM((T * M,), jnp.int32)] * 2,
            [pltpu.VMEM((T * M,), jnp.int32)] * 2,
            [pltpu.VMEM((T * M,), jnp.int32)] * 2,
            [pltpu.VMEM((T, D), jnp.float32)] * 2,
            [pltpu.VMEM((T * M, D), jnp.bfloat16)] * 2,
            [pltpu.SemaphoreType.DMA] * 2,
            [pltpu.SemaphoreType.DMA] * 2,
            [pltpu.SemaphoreType.DMA] * 2,
        ],
        **_SC_PARAMS,
    )
    def k(ids_hbm, chart_hbm, rot_hbm, tok_hbm, out_hbm,
          ids_v, cidx, chars, rc, acc, rb, sem_c, sem_r, sem_t):
        wid = lax.axis_index("s") * info.num_cores + lax.axis_index("c")
        row = wid // wpr
        s0 = (wid % wpr) * per_w
        pltpu.sync_copy(ids_hbm.at[row, pl.ds(s0, per_w)], ids_v)

        def fire_char(c, p):
            v = ids_v[pl.ds(c * T, T)]
            for m in range(M):
                cidx[p][pl.ds(m * T, T)] = v + m * V
            for i in range(NS):
                sl = pl.ds(i * 128, 128)
                pltpu.async_copy(chart_hbm.at[cidx[p].at[sl]], chars[p].at[sl],
                                 sem_c[p])

        def fire_tok(c, p):
            pltpu.async_copy(tok_hbm.at[ids_v.at[pl.ds(c * T, T)]], acc[p],
                             sem_t[p])

        def fire_rot(c, p):
            for i in range(NS):
                sl = pl.ds(i * 128, 128)
                pltpu.make_async_copy(chart_hbm.at[cidx[p].at[sl]],
                                      chars[p].at[sl], sem_c[p]).wait()
            for m in range(M):
                sl = pl.ds(m * T, T)
                rc[p][sl] = chars[p][sl] + m * C
            for i in range(NS):
                sl = pl.ds(i * 128, 128)
                pltpu.async_copy(rot_hbm.at[rc[p].at[sl]], rb[p].at[sl],
                                 sem_r[p])

        def drain_accum(c, p):
            for i in range(NS):
                sl = pl.ds(i * 128, 128)
                pltpu.make_async_copy(rot_hbm.at[rc[p].at[sl]], rb[p].at[sl],
                                      sem_r[p]).wait()
            pltpu.make_async_copy(tok_hbm.at[ids_v.at[pl.ds(c * T, T)]],
                                  acc[p], sem_t[p]).wait()

            def tok_body(j, carry2):
                for g in range(D // 32):
                    a = acc[p][j, pl.ds(g * 32, 16)]
                    bb = acc[p][j, pl.ds(g * 32 + 16, 16)]
                    for m in range(M):
                        raw = rb[p][m * T + j, pl.ds(g * 32, 32)]
                        x, y = plsc.unpack(raw,
                                           format=plsc.PackFormat.INTERLEAVED)
                        a = a + x
                        bb = bb + y
                    acc[p][j, pl.ds(g * 32, 16)] = a
                    acc[p][j, pl.ds(g * 32 + 16, 16)] = bb
                return carry2

            lax.fori_loop(0, T, tok_body, 0)
            pltpu.sync_copy(acc[p], out_hbm.at[row, pl.ds(s0 + c * T, T), :])

        fire_char(0, 0)
        fire_tok(0, 0)
        fire_char(1, 1)
        fire_tok(1, 1)
        fire_rot(0, 0)

        def pair_body(c2, carry):
            c = c2 * 2
            for p in (0, 1):
                cc = c + p
                q = 1 - p
                fire_rot(cc + 1, q)
                fire_char(cc + 2, p)
                drain_accum(cc, p)
                fire_tok(cc + 2, p)
            return carry

        lax.fori_loop(0, (nchunk - 2) // 2, pair_body, 0)

        fire_rot(nchunk - 1, 1)
        drain_accum(nchunk - 2, 0)
        drain_accum(nchunk - 1, 1)

    return k(ids, char_table, rot, tok_emb)


def kernel(input, char_table, char_emb, tok_emb):
    rot = _rot_table(char_emb)
    return _sc_charsum(input, char_table.T.reshape(-1), rot, tok_emb)


# speedup vs baseline: 1.6233x; 1.0720x over previous
import functools
import math

import jax
import jax.numpy as jnp
from jax import lax
from jax.experimental import pallas as pl
from jax.experimental.pallas import tpu as pltpu
from jax.experimental.pallas import tpu_sc as plsc

D = 128
M = 16
C = 256
ROPE_BASE = 10000.0


def _rot_table_body(emb_ref, out_ref):
    e = emb_ref[...]
    col = lax.broadcasted_iota(jnp.int32, (1, D), 1)
    u = col % 32
    ncol = (col - u) + (u % 2) * 16 + u // 2
    two_k = (ncol - (ncol % 2)).astype(jnp.float32)
    freq = jnp.exp(two_k * (-math.log(ROPE_BASE) / D))
    sign = jnp.where((ncol % 2) == 1, 1.0, -1.0)
    rows = lax.broadcasted_iota(jnp.int32, (D, D), 0)
    p1 = (rows == ncol).astype(jnp.float32)
    p2 = (rows == (ncol ^ 1)).astype(jnp.float32)
    ep = jnp.dot(e, p1, preferred_element_type=jnp.float32)
    es = jnp.dot(e, p2, preferred_element_type=jnp.float32)
    for m in range(M):
        ang = m * freq
        cosr = jnp.cos(ang)
        sinr = jnp.sin(ang) * sign
        out_ref[pl.ds(m * C, C), :] = (ep * cosr + es * sinr).astype(
            jnp.bfloat16)


def _rot_table(char_emb):
    return pl.pallas_call(
        _rot_table_body,
        out_shape=jax.ShapeDtypeStruct((M * C, D), jnp.bfloat16),
    )(char_emb)


_SC_PARAMS = dict(
    compiler_params=pltpu.CompilerParams(use_tc_tiling_on_sc=False,
                                         needs_layout_passes=False,
                                         disable_bounds_checks=True),
)


def _sc_info():
    info = plsc.get_sparse_core_info()
    return plsc.VectorSubcoreMesh(core_axis_name="c", subcore_axis_name="s"), info


def _sc_charsum(ids, char_table, rot, tok_emb):
    b, s = ids.shape
    n = b * s
    mesh, info = _sc_info()
    nw = info.num_cores * info.num_subcores
    per_w = n // nw
    wpr = s // per_w
    T = 16
    nchunk = per_w // T
    NS = T * M // 128
    V = char_table.shape[0] // M

    @functools.partial(
        pl.kernel,
        out_type=jax.ShapeDtypeStruct((b, s, D), jnp.float32),
        mesh=mesh,
        scratch_types=[
            pltpu.VMEM((per_w,), jnp.int32),
            [pltpu.VMEM((T * M,), jnp.int32)] * 2,
            [pltpu.VMEM((T * M,), jnp.int32)] * 2,
            [pltpu.VMEM((T * M,), jnp.int32)] * 2,
            [pltpu.VMEM((T, D), jnp.float32)] * 2,
            [pltpu.VMEM((T * M, D), jnp.bfloat16)] * 2,
            [pltpu.SemaphoreType.DMA] * 2,
            [pltpu.SemaphoreType.DMA] * 2,
            [pltpu.SemaphoreType.DMA] * 2,
        ],
        **_SC_PARAMS,
    )
    def k(ids_hbm, chart_hbm, rot_hbm, tok_hbm, out_hbm,
          ids_v, cidx, chars, rc, acc, rb, sem_c, sem_r, sem_t):
        wid = lax.axis_index("s") * info.num_cores + lax.axis_index("c")
        row = wid // wpr
        s0 = (wid % wpr) * per_w
        pltpu.sync_copy(ids_hbm.at[row, pl.ds(s0, per_w)], ids_v)

        def fire_char(c, p):
            v = ids_v[pl.ds(c * T, T)]
            for m in range(M):
                cidx[p][pl.ds(m * T, T)] = v + m * V
            for i in range(NS):
                sl = pl.ds(i * 128, 128)
                pltpu.async_copy(chart_hbm.at[cidx[p].at[sl]], chars[p].at[sl],
                                 sem_c[p])

        def fire_tok(c, p):
            pltpu.async_copy(tok_hbm.at[ids_v.at[pl.ds(c * T, T)]], acc[p],
                             sem_t[p])

        def fire_rot(c, p):
            for i in range(NS):
                sl = pl.ds(i * 128, 128)
                pltpu.make_async_copy(chart_hbm.at[cidx[p].at[sl]],
                                      chars[p].at[sl], sem_c[p]).wait()
            for m in range(M):
                sl = pl.ds(m * T, T)
                rc[p][sl] = chars[p][sl] + m * C
            for i in range(NS):
                sl = pl.ds(i * 128, 128)
                pltpu.async_copy(rot_hbm.at[rc[p].at[sl]], rb[p].at[sl],
                                 sem_r[p])

        def drain_accum(c, p):
            for i in range(NS):
                sl = pl.ds(i * 128, 128)
                pltpu.make_async_copy(rot_hbm.at[rc[p].at[sl]], rb[p].at[sl],
                                      sem_r[p]).wait()
            pltpu.make_async_copy(tok_hbm.at[ids_v.at[pl.ds(c * T, T)]],
                                  acc[p], sem_t[p]).wait()

            def tok_body(j, carry2):
                for g in range(D // 32):
                    a = acc[p][j, pl.ds(g * 32, 16)]
                    bb = acc[p][j, pl.ds(g * 32 + 16, 16)]
                    for m in range(0, M, 2):
                        r0 = rb[p][m * T + j, pl.ds(g * 32, 32)]
                        r1 = rb[p][(m + 1) * T + j, pl.ds(g * 32, 32)]
                        x, y = plsc.unpack(r0 + r1,
                                           format=plsc.PackFormat.INTERLEAVED)
                        a = a + x
                        bb = bb + y
                    acc[p][j, pl.ds(g * 32, 16)] = a
                    acc[p][j, pl.ds(g * 32 + 16, 16)] = bb
                return carry2

            lax.fori_loop(0, T, tok_body, 0)
            pltpu.sync_copy(acc[p], out_hbm.at[row, pl.ds(s0 + c * T, T), :])

        fire_char(0, 0)
        fire_tok(0, 0)
        fire_char(1, 1)
        fire_tok(1, 1)
        fire_rot(0, 0)

        def pair_body(c2, carry):
            c = c2 * 2
            for p in (0, 1):
                cc = c + p
                q = 1 - p
                fire_rot(cc + 1, q)
                fire_char(cc + 2, p)
                drain_accum(cc, p)
                fire_tok(cc + 2, p)
            return carry

        lax.fori_loop(0, (nchunk - 2) // 2, pair_body, 0)

        fire_rot(nchunk - 1, 1)
        drain_accum(nchunk - 2, 0)
        drain_accum(nchunk - 1, 1)

    return k(ids, char_table, rot, tok_emb)


def kernel(input, char_table, char_emb, tok_emb):
    rot = _rot_table(char_emb)
    return _sc_charsum(input, char_table.T.reshape(-1), rot, tok_emb)
